# baseline (speedup 1.0000x reference)
import functools

import jax
import jax.numpy as jnp
from jax import lax
from jax.experimental import pallas as pl
from jax.experimental.pallas import tpu as pltpu
from jax.experimental.pallas import tpu_sc as plsc

N = 10000
E = 320000
D_IN = 128
H = 256
G = 64

NC = 2
NS = 16
L = 16

K = 64
SUPER = 8
OUTER = 40
CHUNKS = SUPER * OUTER
E_PAD = NS * CHUNKS * K
OUTER_D = 20
CHUNKS_D = SUPER * OUTER_D
ACC_ROWS = 10240
ROWS_PER_TILE = ACC_ROWS // NS

_mesh = functools.partial(
    plsc.VectorSubcoreMesh, core_axis_name="c", subcore_axis_name="s"
)


def _deg_body(dst_hbm, out_hbm, idx_d, ones_rows, zbuf, acc):
  c = lax.axis_index("c")
  s = lax.axis_index("s")
  base = (s * NC + c) * OUTER_D

  ones = jnp.full((L,), 1.0, jnp.float32)
  zero = jnp.zeros((L,), jnp.float32)
  for r in range(K):
    for k in range(128 // L):
      ones_rows[r, pl.ds(k * L, L)] = ones
  for r in range(32):
    for k in range(128 // L):
      zbuf[r, pl.ds(k * L, L)] = zero

  def zbody(j, _):
    pltpu.sync_copy(zbuf, acc.at[pl.ds(s * ROWS_PER_TILE + j * 32, 32)])
    return 0

  lax.fori_loop(0, ROWS_PER_TILE // 32, zbody, 0)
  plsc.subcore_barrier()

  def body(o, _):
    pltpu.sync_copy(dst_hbm.at[base + o], idx_d)
    for j in range(SUPER):
      pltpu.sync_copy(ones_rows, acc.at[idx_d.at[j]], add=True)
    return 0

  lax.fori_loop(0, OUTER_D, body, 0)
  plsc.subcore_barrier()

  pltpu.sync_copy(acc.at[pl.ds(s * ROWS_PER_TILE, ROWS_PER_TILE)],
                  out_hbm.at[c].at[pl.ds(s * ROWS_PER_TILE, ROWS_PER_TILE)])


_deg_kernel = pl.kernel(
    _deg_body,
    out_type=jax.ShapeDtypeStruct((NC, ACC_ROWS, 128), jnp.float32),
    mesh=_mesh(),
    scratch_types=[
        pltpu.VMEM((SUPER, K), jnp.int32),
        pltpu.VMEM((K, 128), jnp.float32),
        pltpu.VMEM((32, 128), jnp.float32),
        pltpu.VMEM_SHARED((ACC_ROWS, 128), jnp.float32),
    ],
)


def _agg_body(core_split, x2_hbm, src_hbm, dst_hbm, out_hbm,
              idx_s, idx_d, rows0, rows1, rows2, rows3, zbuf, acc, sem):
  c = lax.axis_index("c")
  s = lax.axis_index("s")
  if core_split:
    base, outer, xi = s * OUTER, OUTER, c
  else:
    base, outer, xi = (s * NC + c) * OUTER_D, OUTER_D, 0

  for r in range(32):
    for k in range(128 // L):
      zbuf[r, pl.ds(k * L, L)] = jnp.zeros((L,), jnp.float32)

  def zbody(j, _):
    pltpu.sync_copy(zbuf, acc.at[pl.ds(s * ROWS_PER_TILE + j * 32, 32)])
    return 0

  lax.fori_loop(0, ROWS_PER_TILE // 32, zbody, 0)
  plsc.subcore_barrier()

  xc = x2_hbm.at[xi]
  rows = [rows0, rows1, rows2, rows3]
  NB = len(rows)
  gsem, ssem = sem

  def body(o, _):
    pltpu.sync_copy(src_hbm.at[base + o], idx_s)
    pltpu.sync_copy(dst_hbm.at[base + o], idx_d)
    dg = [None] * NB
    ds = [None] * NB
    for j in range(NB - 1):
      dg[j] = pltpu.async_copy(xc.at[idx_s.at[j]], rows[j], gsem)
    for j in range(SUPER):
      dg[j % NB].wait()
      nxt = j + NB - 1
      if nxt < SUPER:
        if ds[nxt % NB] is not None:
          ds[nxt % NB].wait()
        dg[nxt % NB] = pltpu.async_copy(
            xc.at[idx_s.at[nxt]], rows[nxt % NB], gsem)
      ds[j % NB] = pltpu.async_copy(rows[j % NB], acc.at[idx_d.at[j]], ssem,
                                    add=True)
    for j in range(SUPER - NB, SUPER):
      ds[j % NB].wait()
    return 0

  lax.fori_loop(0, outer, body, 0)
  plsc.subcore_barrier()

  pltpu.sync_copy(acc.at[pl.ds(s * ROWS_PER_TILE, ROWS_PER_TILE)],
                  out_hbm.at[c].at[pl.ds(s * ROWS_PER_TILE, ROWS_PER_TILE)])


def _make_agg_kernel(core_split):
  return pl.kernel(
      functools.partial(_agg_body, core_split),
      out_type=jax.ShapeDtypeStruct((NC, ACC_ROWS, 128), jnp.float32),
      mesh=_mesh(),
      scratch_types=[
          pltpu.VMEM((SUPER, K), jnp.int32),
          pltpu.VMEM((SUPER, K), jnp.int32),
          pltpu.VMEM((K, 128), jnp.float32),
          pltpu.VMEM((K, 128), jnp.float32),
          pltpu.VMEM((K, 128), jnp.float32),
          pltpu.VMEM((K, 128), jnp.float32),
          pltpu.VMEM((32, 128), jnp.float32),
          pltpu.VMEM_SHARED((ACC_ROWS, 128), jnp.float32),
          (pltpu.SemaphoreType.DMA, pltpu.SemaphoreType.DMA),
      ],
  )


_agg_kernel_128 = _make_agg_kernel(True)
_agg1_kernel = _make_agg_kernel(False)


def _tc_block_body(with_res, agg2, h, deg2, wl, bl, wr, gamma, beta,
                   wres, bres, out):
  if with_res:
    agg = agg2[0, :N] + agg2[1, :N]
  else:
    agg = jnp.concatenate([agg2[0, :N], agg2[1, :N]], axis=1)
  deg = jnp.maximum(deg2[0, :] + deg2[1, :], 1.0)
  aggd = agg * (1.0 / deg)[:, None]
  x = h[...]
  o = (jnp.dot(aggd, wl[...], preferred_element_type=jnp.float32)
       + jnp.dot(x, wr[...], preferred_element_type=jnp.float32)
       + bl[...][None, :])
  mean = jnp.mean(o, axis=0)
  cent = o - mean[None, :]
  var = jnp.mean(cent * cent, axis=0)
  o = cent * lax.rsqrt(var + 1e-5)[None, :] * gamma[...][None, :] + beta[...][None, :]
  o = jnp.maximum(o, 0.0)
  if with_res:
    o = o + jnp.dot(x, wres[...], preferred_element_type=jnp.float32) \
        + bres[...][None, :]
  else:
    o = o + x
  out[0] = o[:, :H // 2]
  out[1] = o[:, H // 2:]


def _tc_block(agg2, h, deg2, p, with_res):
  wres = p["W_res"] if with_res else p["W_l"]
  bres = p["b_res"] if with_res else p["b_l"]
  return pl.pallas_call(
      functools.partial(_tc_block_body, with_res),
      out_shape=jax.ShapeDtypeStruct((NC, N, H // 2), jnp.float32),
      name="tc_block",
  )(agg2, h, deg2, p["W_l"], p["b_l"], p["W_r"], p["gamma"], p["beta"],
    wres, bres)


def _pool_body(h2, batch, wout, bout, out):
  h = jnp.concatenate([h2[0], h2[1]], axis=1)
  gid = lax.broadcasted_iota(jnp.int32, (N, G), 1)
  onehot = (batch[...] == gid).astype(jnp.float32)
  sums = lax.dot_general(onehot, h, (((0,), (0,)), ((), ())),
                         preferred_element_type=jnp.float32)
  cnt = jnp.sum(onehot, axis=0)
  pooled = sums * (1.0 / jnp.maximum(cnt, 1.0))[:, None]
  out[...] = jnp.dot(pooled, wout[...], preferred_element_type=jnp.float32) \
      + bout[...][None, :]


_pool_kernel = pl.pallas_call(
    _pool_body,
    out_shape=jax.ShapeDtypeStruct((G, 1), jnp.float32),
)


@jax.jit
def kernel(x, edge_index, batch, params):
  perm = jnp.argsort(edge_index[0])
  src = edge_index[0, perm]
  dst = edge_index[1, perm]
  pad = E_PAD - E
  src_t = jnp.concatenate([src, jnp.zeros((pad,), jnp.int32)]) \
      .reshape(NC * NS * OUTER_D, SUPER, K)
  dst_t = jnp.concatenate([dst, jnp.full((pad,), N, jnp.int32)]) \
      .reshape(NC * NS * OUTER_D, SUPER, K)
  deg2 = _deg_kernel(dst_t)[:, :N, 0]

  a = _agg1_kernel(x[None], src_t, dst_t)
  h = _tc_block(a, x, deg2, params["b1"], True)

  for name in ("b2", "b3", "b4"):
    a = _agg_kernel_128(h, src_t, dst_t)
    hm = jnp.concatenate([h[0], h[1]], axis=1)
    h = _tc_block(a, hm, deg2, params[name], False)

  batch2 = batch.reshape(N, 1)
  return _pool_kernel(h, batch2, params["W_out"], params["b_out"])


# speedup vs baseline: 1.4316x; 1.4316x over previous
import functools

import jax
import jax.numpy as jnp
from jax import lax
from jax.experimental import pallas as pl
from jax.experimental.pallas import tpu as pltpu
from jax.experimental.pallas import tpu_sc as plsc

N = 10000
E = 320000
D_IN = 128
H = 256
G = 64

NC = 2
NS = 16
L = 16

K = 64
SUPER = 8
OUTER = 40
CHUNKS = SUPER * OUTER
E_PAD = NS * CHUNKS * K
OUTER_D = 20
CHUNKS_D = SUPER * OUTER_D
ACC_ROWS = 10240
ROWS_PER_TILE = ACC_ROWS // NS

_mesh = functools.partial(
    plsc.VectorSubcoreMesh, core_axis_name="c", subcore_axis_name="s"
)


def _deg_body(dst_hbm, out_hbm, idx_d, ones_rows, zbuf, acc):
  c = lax.axis_index("c")
  s = lax.axis_index("s")
  base = (s * NC + c) * OUTER_D

  ones = jnp.full((L,), 1.0, jnp.float32)
  zero = jnp.zeros((L,), jnp.float32)
  for r in range(K):
    for k in range(128 // L):
      ones_rows[r, pl.ds(k * L, L)] = ones
  for r in range(32):
    for k in range(128 // L):
      zbuf[r, pl.ds(k * L, L)] = zero

  def zbody(j, _):
    pltpu.sync_copy(zbuf, acc.at[pl.ds(s * ROWS_PER_TILE + j * 32, 32)])
    return 0

  lax.fori_loop(0, ROWS_PER_TILE // 32, zbody, 0)
  plsc.subcore_barrier()

  def body(o, _):
    pltpu.sync_copy(dst_hbm.at[base + o], idx_d)
    for j in range(SUPER):
      pltpu.sync_copy(ones_rows, acc.at[idx_d.at[j]], add=True)
    return 0

  lax.fori_loop(0, OUTER_D, body, 0)
  plsc.subcore_barrier()

  pltpu.sync_copy(acc.at[pl.ds(s * ROWS_PER_TILE, ROWS_PER_TILE)],
                  out_hbm.at[c].at[pl.ds(s * ROWS_PER_TILE, ROWS_PER_TILE)])


_deg_kernel = pl.kernel(
    _deg_body,
    out_type=jax.ShapeDtypeStruct((NC, ACC_ROWS, 128), jnp.float32),
    mesh=_mesh(),
    scratch_types=[
        pltpu.VMEM((SUPER, K), jnp.int32),
        pltpu.VMEM((K, 128), jnp.float32),
        pltpu.VMEM((32, 128), jnp.float32),
        pltpu.VMEM_SHARED((ACC_ROWS, 128), jnp.float32),
    ],
)


def _agg_body(core_split, x2_hbm, src_hbm, dst_hbm, out_hbm,
              idx_s, idx_d, rows0, rows1, rows2, rows3, zbuf, acc, sem):
  c = lax.axis_index("c")
  s = lax.axis_index("s")
  if core_split:
    base, outer, xi = s * OUTER, OUTER, c
  else:
    base, outer, xi = (s * NC + c) * OUTER_D, OUTER_D, 0

  for r in range(32):
    for k in range(128 // L):
      zbuf[r, pl.ds(k * L, L)] = jnp.zeros((L,), jnp.float32)

  def zbody(j, _):
    pltpu.sync_copy(zbuf, acc.at[pl.ds(s * ROWS_PER_TILE + j * 32, 32)])
    return 0

  lax.fori_loop(0, ROWS_PER_TILE // 32, zbody, 0)
  plsc.subcore_barrier()

  xc = x2_hbm.at[xi]
  rows = [rows0, rows1, rows2, rows3]
  NB = len(rows)
  gsem, ssem = sem

  def body(o, _):
    pltpu.sync_copy(src_hbm.at[base + o], idx_s)
    pltpu.sync_copy(dst_hbm.at[base + o], idx_d)
    dg = [None] * NB
    ds = [None] * NB
    for j in range(NB - 1):
      dg[j] = pltpu.async_copy(xc.at[idx_s.at[j]], rows[j], gsem)
    for j in range(SUPER):
      dg[j % NB].wait()
      nxt = j + NB - 1
      if nxt < SUPER:
        if ds[nxt % NB] is not None:
          ds[nxt % NB].wait()
        dg[nxt % NB] = pltpu.async_copy(
            xc.at[idx_s.at[nxt]], rows[nxt % NB], gsem)
      ds[j % NB] = pltpu.async_copy(rows[j % NB], acc.at[idx_d.at[j]], ssem,
                                    add=True)
    for j in range(SUPER - NB, SUPER):
      ds[j % NB].wait()
    return 0

  lax.fori_loop(0, outer, body, 0)
  plsc.subcore_barrier()

  pltpu.sync_copy(acc.at[pl.ds(s * ROWS_PER_TILE, ROWS_PER_TILE)],
                  out_hbm.at[c].at[pl.ds(s * ROWS_PER_TILE, ROWS_PER_TILE)])


def _make_agg_kernel(core_split):
  return pl.kernel(
      functools.partial(_agg_body, core_split),
      out_type=jax.ShapeDtypeStruct((NC, ACC_ROWS, 128), jnp.float32),
      mesh=_mesh(),
      scratch_types=[
          pltpu.VMEM((SUPER, K), jnp.int32),
          pltpu.VMEM((SUPER, K), jnp.int32),
          pltpu.VMEM((K, 128), jnp.float32),
          pltpu.VMEM((K, 128), jnp.float32),
          pltpu.VMEM((K, 128), jnp.float32),
          pltpu.VMEM((K, 128), jnp.float32),
          pltpu.VMEM((32, 128), jnp.float32),
          pltpu.VMEM_SHARED((ACC_ROWS, 128), jnp.float32),
          (pltpu.SemaphoreType.DMA, pltpu.SemaphoreType.DMA),
      ],
  )


_agg_kernel_128 = _make_agg_kernel(True)
_agg1_kernel = _make_agg_kernel(False)


def _tc_block_body(with_res, agg2, h, deg2, wl, bl, wr, gamma, beta,
                   wres, bres, out):
  if with_res:
    agg = agg2[0, :N] + agg2[1, :N]
  else:
    agg = jnp.concatenate([agg2[0, :N], agg2[1, :N]], axis=1)
  deg = jnp.maximum(deg2[0, :] + deg2[1, :], 1.0)
  aggd = agg * (1.0 / deg)[:, None]
  x = h[...]
  o = (jnp.dot(aggd, wl[...], preferred_element_type=jnp.float32)
       + jnp.dot(x, wr[...], preferred_element_type=jnp.float32)
       + bl[...][None, :])
  mean = jnp.mean(o, axis=0)
  cent = o - mean[None, :]
  var = jnp.mean(cent * cent, axis=0)
  o = cent * lax.rsqrt(var + 1e-5)[None, :] * gamma[...][None, :] + beta[...][None, :]
  o = jnp.maximum(o, 0.0)
  if with_res:
    o = o + jnp.dot(x, wres[...], preferred_element_type=jnp.float32) \
        + bres[...][None, :]
  else:
    o = o + x
  out[0] = o[:, :H // 2]
  out[1] = o[:, H // 2:]


def _tc_block(agg2, h, deg2, p, with_res):
  wres = p["W_res"] if with_res else p["W_l"]
  bres = p["b_res"] if with_res else p["b_l"]
  return pl.pallas_call(
      functools.partial(_tc_block_body, with_res),
      out_shape=jax.ShapeDtypeStruct((NC, N, H // 2), jnp.float32),
      name="tc_block",
  )(agg2, h, deg2, p["W_l"], p["b_l"], p["W_r"], p["gamma"], p["beta"],
    wres, bres)


def _pool_body(h2, batch, wout, bout, out):
  h = jnp.concatenate([h2[0], h2[1]], axis=1)
  gid = lax.broadcasted_iota(jnp.int32, (N, G), 1)
  onehot = (batch[...] == gid).astype(jnp.float32)
  sums = lax.dot_general(onehot, h, (((0,), (0,)), ((), ())),
                         preferred_element_type=jnp.float32)
  cnt = jnp.sum(onehot, axis=0)
  pooled = sums * (1.0 / jnp.maximum(cnt, 1.0))[:, None]
  out[...] = jnp.dot(pooled, wout[...], preferred_element_type=jnp.float32) \
      + bout[...][None, :]


_pool_kernel = pl.pallas_call(
    _pool_body,
    out_shape=jax.ShapeDtypeStruct((G, 1), jnp.float32),
)


@jax.jit
def kernel(x, edge_index, batch, params):
  src = edge_index[0]
  dst = edge_index[1]
  pad = E_PAD - E
  src_t = jnp.concatenate([src, jnp.zeros((pad,), jnp.int32)]) \
      .reshape(NC * NS * OUTER_D, SUPER, K)
  dst_t = jnp.concatenate([dst, jnp.full((pad,), N, jnp.int32)]) \
      .reshape(NC * NS * OUTER_D, SUPER, K)
  deg2 = _deg_kernel(dst_t)[:, :N, 0]

  a = _agg1_kernel(x[None], src_t, dst_t)
  h = _tc_block(a, x, deg2, params["b1"], True)

  for name in ("b2", "b3", "b4"):
    a = _agg_kernel_128(h, src_t, dst_t)
    hm = jnp.concatenate([h[0], h[1]], axis=1)
    h = _tc_block(a, hm, deg2, params[name], False)

  batch2 = batch.reshape(N, 1)
  return _pool_kernel(h, batch2, params["W_out"], params["b_out"])
